# baseline (speedup 1.0000x reference)
import functools

import jax
import jax.numpy as jnp
from jax import lax
from jax.experimental import pallas as pl
from jax.experimental.pallas import tpu as pltpu
from jax.experimental.pallas import tpu_sc as plsc

N_ROWS = 1048576
NT = N_ROWS // 128
NW = 32
TILES_W = NT // NW
G = 64
NCH = TILES_W // G
L = 16


def _sc_rejection(eps3, prob, consts):
    mesh = plsc.VectorSubcoreMesh(core_axis_name="c", subcore_axis_name="s")

    @functools.partial(
        pl.kernel,
        mesh=mesh,
        compiler_params=pltpu.CompilerParams(
            needs_layout_passes=False, use_tc_tiling_on_sc=False),
        out_type=jax.ShapeDtypeStruct((NT, 2, 128), jnp.float32),
        scratch_types=[
            pltpu.VMEM((2, G, 2, 128), jnp.float32),
            pltpu.VMEM((2, G, 2, 128), jnp.float32),
            pltpu.VMEM((2, G * 128), jnp.float32),
            pltpu.VMEM((64,), jnp.float32),
            pltpu.SemaphoreType.DMA,
            pltpu.SemaphoreType.DMA,
            pltpu.SemaphoreType.DMA,
            pltpu.SemaphoreType.DMA,
            pltpu.SemaphoreType.DMA,
            pltpu.SemaphoreType.DMA,
        ],
    )
    def k(eps_hbm, prob_hbm, consts_hbm, out_hbm, ebuf, obuf, pbuf,
          cbuf, sie0, sie1, sip0, sip1, so0, so1):
        cid = lax.axis_index("c")
        sid = lax.axis_index("s")
        wid = sid * 2 + cid
        pltpu.sync_copy(consts_hbm, cbuf)
        s0 = cbuf[pl.ds(0, L)]
        s1 = cbuf[pl.ds(L, L)]
        t0 = cbuf[pl.ds(2 * L, L)]
        t1 = cbuf[pl.ds(3 * L, L)]
        tile0 = wid * TILES_W
        sie = (sie0, sie1)
        sip = (sip0, sip1)
        so = (so0, so1)

        def start_in(ci):
            b = ci % 2
            tbase = tile0 + ci * G
            cp_e = pltpu.async_copy(
                eps_hbm.at[pl.ds(tbase, G)], ebuf.at[b], sie[b])
            cp_p = pltpu.async_copy(
                prob_hbm.at[pl.ds(tbase * 128, G * 128)], pbuf.at[b], sip[b])
            return cp_e, cp_p

        def compute(ci):
            b = ci % 2

            @plsc.parallel_loop(0, G * 8, unroll=4)
            def body(i):
                t = i >> 3
                w = (i & 7) * L
                e0 = ebuf[b, t, 0, pl.ds(w, L)]
                e1 = ebuf[b, t, 1, pl.ds(w, L)]
                z0 = e0 * s0 + t0
                z1 = e1 * s1 + t1
                tot = z0 * z0 + z1 * z1
                p = pbuf[b, pl.ds(i * L, L)]
                acc = jnp.exp(tot * -0.5) > p
                zero = jnp.zeros((L,), jnp.float32)
                obuf[b, t, 0, pl.ds(w, L)] = jnp.where(acc, z0, zero)
                obuf[b, t, 1, pl.ds(w, L)] = jnp.where(acc, z1, zero)

        def start_out(ci):
            b = ci % 2
            tbase = tile0 + ci * G
            return pltpu.async_copy(
                obuf.at[b], out_hbm.at[pl.ds(tbase, G)], so[b])

        cps_in = {0: start_in(0), 1: start_in(1)}
        cps_out = {}
        for ci in range(NCH):
            cp_e, cp_p = cps_in.pop(ci)
            cp_e.wait()
            cp_p.wait()
            if ci >= 2:
                cps_out.pop(ci - 2).wait()
            compute(ci)
            cps_out[ci] = start_out(ci)
            if ci + 2 < NCH:
                cps_in[ci + 2] = start_in(ci + 2)
        cps_out.pop(NCH - 2).wait()
        cps_out.pop(NCH - 1).wait()

    return k(eps3, prob, consts)


def kernel(eps, prob, prop_scale, prop_shift):
    consts = jnp.repeat(
        jnp.concatenate([prop_scale, prop_shift]).astype(jnp.float32), 16)
    eps3 = eps.reshape(NT, 128, 2).transpose(0, 2, 1)
    out3 = _sc_rejection(eps3, prob, consts)
    return out3.transpose(0, 2, 1).reshape(N_ROWS, 2)


# speedup vs baseline: 1.0203x; 1.0203x over previous
import functools

import jax
import jax.numpy as jnp
from jax import lax
from jax.experimental import pallas as pl
from jax.experimental.pallas import tpu as pltpu
from jax.experimental.pallas import tpu_sc as plsc

N_ROWS = 1048576
NT = N_ROWS // 128
NW = 32
TILES_W = NT // NW
G = 64
NCH = TILES_W // G
L = 16


def _sc_rejection(eps3, prob, consts):
    mesh = plsc.VectorSubcoreMesh(core_axis_name="c", subcore_axis_name="s")

    @functools.partial(
        pl.kernel,
        mesh=mesh,
        compiler_params=pltpu.CompilerParams(
            needs_layout_passes=False, use_tc_tiling_on_sc=False,
            skip_device_barrier=True, disable_bounds_checks=True,
            disable_semaphore_checks=True),
        out_type=jax.ShapeDtypeStruct((NT, 2, 128), jnp.float32),
        scratch_types=[
            pltpu.VMEM((2, G, 2, 128), jnp.float32),
            pltpu.VMEM((2, G, 2, 128), jnp.float32),
            pltpu.VMEM((2, G * 128), jnp.float32),
            pltpu.VMEM((64,), jnp.float32),
            pltpu.SemaphoreType.DMA,
            pltpu.SemaphoreType.DMA,
            pltpu.SemaphoreType.DMA,
            pltpu.SemaphoreType.DMA,
            pltpu.SemaphoreType.DMA,
            pltpu.SemaphoreType.DMA,
        ],
    )
    def k(eps_hbm, prob_hbm, consts_hbm, out_hbm, ebuf, obuf, pbuf,
          cbuf, sie0, sie1, sip0, sip1, so0, so1):
        cid = lax.axis_index("c")
        sid = lax.axis_index("s")
        wid = sid * 2 + cid
        pltpu.sync_copy(consts_hbm, cbuf)
        s0 = cbuf[pl.ds(0, L)]
        s1 = cbuf[pl.ds(L, L)]
        t0 = cbuf[pl.ds(2 * L, L)]
        t1 = cbuf[pl.ds(3 * L, L)]
        tile0 = wid * TILES_W
        sie = (sie0, sie1)
        sip = (sip0, sip1)
        so = (so0, so1)

        def start_in(ci):
            b = ci % 2
            tbase = tile0 + ci * G
            cp_e = pltpu.async_copy(
                eps_hbm.at[pl.ds(tbase, G)], ebuf.at[b], sie[b])
            cp_p = pltpu.async_copy(
                prob_hbm.at[pl.ds(tbase * 128, G * 128)], pbuf.at[b], sip[b])
            return cp_e, cp_p

        def compute(ci):
            b = ci % 2

            @plsc.parallel_loop(0, G * 8, unroll=2)
            def body(i):
                t = i >> 3
                w = (i & 7) * L
                e0 = ebuf[b, t, 0, pl.ds(w, L)]
                e1 = ebuf[b, t, 1, pl.ds(w, L)]
                z0 = e0 * s0 + t0
                z1 = e1 * s1 + t1
                tot = z0 * z0 + z1 * z1
                p = pbuf[b, pl.ds(i * L, L)]
                acc = jnp.exp(tot * -0.5) > p
                zero = jnp.zeros((L,), jnp.float32)
                obuf[b, t, 0, pl.ds(w, L)] = jnp.where(acc, z0, zero)
                obuf[b, t, 1, pl.ds(w, L)] = jnp.where(acc, z1, zero)

        def start_out(ci):
            b = ci % 2
            tbase = tile0 + ci * G
            return pltpu.async_copy(
                obuf.at[b], out_hbm.at[pl.ds(tbase, G)], so[b])

        cps_in = {0: start_in(0), 1: start_in(1)}
        cps_out = {}
        for ci in range(NCH):
            cp_e, cp_p = cps_in.pop(ci)
            cp_e.wait()
            cp_p.wait()
            if ci >= 2:
                cps_out.pop(ci - 2).wait()
            compute(ci)
            cps_out[ci] = start_out(ci)
            if ci + 2 < NCH:
                cps_in[ci + 2] = start_in(ci + 2)
        cps_out.pop(NCH - 2).wait()
        cps_out.pop(NCH - 1).wait()

    return k(eps3, prob, consts)


def kernel(eps, prob, prop_scale, prop_shift):
    consts = jnp.repeat(
        jnp.concatenate([prop_scale, prop_shift]).astype(jnp.float32), 16)
    eps3 = eps.reshape(NT, 128, 2).transpose(0, 2, 1)
    out3 = _sc_rejection(eps3, prob, consts)
    return out3.transpose(0, 2, 1).reshape(N_ROWS, 2)
